# baseline (speedup 1.0000x reference)
import functools

import jax
import jax.numpy as jnp
from jax import lax
from jax.experimental import pallas as pl
from jax.experimental.pallas import tpu as pltpu
from jax.experimental.pallas import tpu_sc as plsc

_B = 10000
_K = 32
_D = 128

_NC = 2
_NS = 16
_NW = _NC * _NS
_CH = 128

_PARTS = ((3200, 25), (3200, 25), (3200, 25), (400, 5))
_C2 = 3
_N2_PAD = _NW * _C2 * _CH

_BB = 200


def _sc_gather_body(c1, with_self, table_h, idx1_h, idx2_h, out1_h, out2_h,
                    idx1_v, idx2_v, bufs, gsems, osems):
    wid = lax.axis_index("s") * _NC + lax.axis_index("c")
    pltpu.sync_copy(idx1_h.at[wid], idx1_v)
    if with_self:
        pltpu.sync_copy(idx2_h.at[wid], idx2_v)

    def start_g(idx_v, j, b):
        pltpu.make_async_copy(
            table_h.at[idx_v.at[j]], bufs.at[b], gsems.at[b]).start()

    def wait_g(b):
        pltpu.make_async_copy(
            table_h.at[idx1_v.at[0]], bufs.at[b], gsems.at[b]).wait()

    def start_s(out_h, row0, b):
        pltpu.make_async_copy(
            bufs.at[b], out_h.at[pl.ds(row0, _CH)], osems.at[b]).start()

    def wait_s(b):
        pltpu.make_async_copy(
            bufs.at[b], out1_h.at[pl.ds(0, _CH)], osems.at[b]).wait()

    base1 = wid * c1 * _CH

    for b in range(3):
        start_g(idx1_v, b, b)

    @pl.loop(0, c1 // 5)
    def _round(t):
        for b in range(5):
            j = 5 * t + b
            wait_g(b)
            start_s(out1_h, base1 + j * _CH, b)
            b2 = (b + 3) % 5

            @pl.when(j + 3 < c1)
            def _():
                @pl.when(j >= 2)
                def _():
                    wait_s(b2)

                start_g(idx1_v, j + 3, b2)

    for b in range(5):
        wait_s(b)

    if with_self:
        base2 = wid * _C2 * _CH
        for j in range(_C2):
            pltpu.async_copy(
                table_h.at[idx2_v.at[j]], bufs.at[j], gsems.at[j]).wait()
            start_s(out2_h, base2 + j * _CH, j)
        for j in range(_C2):
            pltpu.make_async_copy(
                bufs.at[j], out2_h.at[pl.ds(0, _CH)], osems.at[j]).wait()


_MESH = plsc.VectorSubcoreMesh(core_axis_name="c", subcore_axis_name="s")


def _scratch(c1):
    return [
        pltpu.VMEM((c1, _CH), jnp.int32),
        pltpu.VMEM((_C2, _CH), jnp.int32),
        pltpu.VMEM((5, _CH, _D), jnp.float32),
        pltpu.SemaphoreType.DMA((5,)),
        pltpu.SemaphoreType.DMA((5,)),
    ]


def _make_sc(c1, with_self):
    n1 = _NW * c1 * _CH

    if with_self:
        def body(table_h, idx1_h, idx2_h, out1_h, out2_h, *scratch):
            _sc_gather_body(c1, True, table_h, idx1_h, idx2_h,
                            out1_h, out2_h, *scratch)

        out_type = (jax.ShapeDtypeStruct((n1, _D), jnp.float32),
                    jax.ShapeDtypeStruct((_N2_PAD, _D), jnp.float32))
    else:
        def body(table_h, idx1_h, out1_h, *scratch):
            idx1_v, idx2_v, bufs, gsems, osems = scratch
            _sc_gather_body(c1, False, table_h, idx1_h, None, out1_h, None,
                            idx1_v, idx2_v, bufs, gsems, osems)

        out_type = jax.ShapeDtypeStruct((n1, _D), jnp.float32)

    k = pl.kernel(body, out_type=out_type, mesh=_MESH,
                  scratch_types=_scratch(c1))
    return k


_SC_KERNELS = tuple(
    _make_sc(c1, p == 0) for p, (_, c1) in enumerate(_PARTS))


def _tc_mlp_body(e3_ref, u_ref, w1t_ref, w1b_ref, w2_ref, w3t_ref,
                 b1_ref, b2_ref, b3_ref, out_ref):
    e3 = e3_ref[...]
    e2 = e3.reshape(_BB * _K, _D).astype(jnp.bfloat16)
    u = u_ref[...].astype(jnp.bfloat16)

    uw = jnp.dot(u, w1b_ref[...], preferred_element_type=jnp.float32)
    uw = uw + b1_ref[...]
    z1 = jnp.dot(e2, w1t_ref[...], preferred_element_type=jnp.float32)
    h1 = jnp.maximum(z1.reshape(_BB, _K, _D) + uw[:, None, :], 0.0)

    h2 = jnp.dot(h1.reshape(_BB * _K, _D).astype(jnp.bfloat16), w2_ref[...],
                 preferred_element_type=jnp.float32)
    h2 = jnp.maximum(h2 + b2_ref[...], 0.0)

    w3row = w3t_ref[...].reshape(1, 1, _D)
    t = jnp.sum(h2.reshape(_BB, _K, _D) * w3row, axis=2, keepdims=True)
    t = t + b3_ref[0, 0]

    m = jnp.max(t, axis=1, keepdims=True)
    p = jnp.exp(t - m)
    s = jnp.sum(p, axis=1, keepdims=True)
    att = p / s

    out_ref[...] = jnp.sum(e3 * att, axis=1)


def _tc_mlp(nodes_p, ublk, e3, u, w1t, w1b, w2, w3t, b1, b2, b3):
    return pl.pallas_call(
        _tc_mlp_body,
        grid=(nodes_p // _BB,),
        in_specs=[
            pl.BlockSpec((_BB, _K, _D), lambda i: (i, 0, 0)),
            pl.BlockSpec((_BB, _D), lambda i: (i + ublk, 0)),
            pl.BlockSpec((_D, _D), lambda i: (0, 0)),
            pl.BlockSpec((_D, _D), lambda i: (0, 0)),
            pl.BlockSpec((_D, _D), lambda i: (0, 0)),
            pl.BlockSpec((1, _D), lambda i: (0, 0)),
            pl.BlockSpec((1, _D), lambda i: (0, 0)),
            pl.BlockSpec((1, _D), lambda i: (0, 0)),
            pl.BlockSpec((1, 1), lambda i: (0, 0)),
        ],
        out_specs=pl.BlockSpec((_BB, _D), lambda i: (i, 0)),
        out_shape=jax.ShapeDtypeStruct((nodes_p, _D), jnp.float32),
    )(e3, u, w1t, w1b, w2, w3t, b1, b2, b3)


def kernel(nodes, to_neighs, u2e, W1, b1, W2, b2, W3, b3):
    nflat = to_neighs.reshape(-1)
    idx2 = jnp.zeros((_N2_PAD,), jnp.int32).at[:_B].set(
        nodes).reshape(_NW, _C2, _CH)

    idx1 = []
    row0 = 0
    for nodes_p, c1 in _PARTS:
        n1 = _NW * c1 * _CH
        nrows = nodes_p * _K
        idx1.append(jnp.zeros((n1,), jnp.int32).at[:nrows].set(
            nflat[row0: row0 + nrows]).reshape(_NW, c1, _CH))
        row0 += nrows

    e_parts = []
    for p, k in enumerate(_SC_KERNELS):
        if p == 0:
            ea, u_rows = k(u2e, idx1[0], idx2)
            e_parts.append(ea)
        else:
            e_parts.append(k(u2e, idx1[p]))

    bf = jnp.bfloat16
    args = (W1[:_D].astype(bf), W1[_D:].astype(bf), W2.astype(bf),
            W3.reshape(1, _D), b1.reshape(1, _D), b2.reshape(1, _D),
            b3.reshape(1, 1))
    outs = []
    ublk = 0
    for (nodes_p, c1), rows in zip(_PARTS, e_parts):
        e3 = rows.reshape(rows.shape[0] // _K, _K, _D)
        outs.append(_tc_mlp(nodes_p, ublk, e3, u_rows, *args))
        ublk += nodes_p // _BB
    return jnp.concatenate(outs, axis=0)


# speedup vs baseline: 1.0398x; 1.0398x over previous
import functools

import jax
import jax.numpy as jnp
from jax import lax
from jax.experimental import pallas as pl
from jax.experimental.pallas import tpu as pltpu
from jax.experimental.pallas import tpu_sc as plsc

_B = 10000
_K = 32
_D = 128

_NC = 2
_NS = 16
_NW = _NC * _NS
_CH = 128

_PARTS = ((3200, 25), (3200, 25), (3600, 30))
_C2 = 3
_N2_PAD = _NW * _C2 * _CH

_BB = 200


def _sc_gather_body(c1, with_self, table_h, idx1_h, idx2_h, out1_h, out2_h,
                    idx1_v, idx2_v, bufs, gsems, osems):
    wid = lax.axis_index("s") * _NC + lax.axis_index("c")
    pltpu.sync_copy(idx1_h.at[wid], idx1_v)
    if with_self:
        pltpu.sync_copy(idx2_h.at[wid], idx2_v)

    def start_g(idx_v, j, b):
        pltpu.make_async_copy(
            table_h.at[idx_v.at[j]], bufs.at[b], gsems.at[b]).start()

    def wait_g(b):
        pltpu.make_async_copy(
            table_h.at[idx1_v.at[0]], bufs.at[b], gsems.at[b]).wait()

    def start_s(out_h, row0, b):
        pltpu.make_async_copy(
            bufs.at[b], out_h.at[pl.ds(row0, _CH)], osems.at[b]).start()

    def wait_s(b):
        pltpu.make_async_copy(
            bufs.at[b], out1_h.at[pl.ds(0, _CH)], osems.at[b]).wait()

    base1 = wid * c1 * _CH

    for b in range(3):
        start_g(idx1_v, b, b)

    @pl.loop(0, c1 // 5)
    def _round(t):
        for b in range(5):
            j = 5 * t + b
            wait_g(b)
            start_s(out1_h, base1 + j * _CH, b)
            b2 = (b + 3) % 5

            @pl.when(j + 3 < c1)
            def _():
                @pl.when(j >= 2)
                def _():
                    wait_s(b2)

                start_g(idx1_v, j + 3, b2)

    for b in range(5):
        wait_s(b)

    if with_self:
        base2 = wid * _C2 * _CH
        for j in range(_C2):
            pltpu.async_copy(
                table_h.at[idx2_v.at[j]], bufs.at[j], gsems.at[j]).wait()
            start_s(out2_h, base2 + j * _CH, j)
        for j in range(_C2):
            pltpu.make_async_copy(
                bufs.at[j], out2_h.at[pl.ds(0, _CH)], osems.at[j]).wait()


_MESH = plsc.VectorSubcoreMesh(core_axis_name="c", subcore_axis_name="s")


def _scratch(c1):
    return [
        pltpu.VMEM((c1, _CH), jnp.int32),
        pltpu.VMEM((_C2, _CH), jnp.int32),
        pltpu.VMEM((5, _CH, _D), jnp.float32),
        pltpu.SemaphoreType.DMA((5,)),
        pltpu.SemaphoreType.DMA((5,)),
    ]


def _make_sc(c1, with_self):
    n1 = _NW * c1 * _CH

    if with_self:
        def body(table_h, idx1_h, idx2_h, out1_h, out2_h, *scratch):
            _sc_gather_body(c1, True, table_h, idx1_h, idx2_h,
                            out1_h, out2_h, *scratch)

        out_type = (jax.ShapeDtypeStruct((n1, _D), jnp.float32),
                    jax.ShapeDtypeStruct((_N2_PAD, _D), jnp.float32))
    else:
        def body(table_h, idx1_h, out1_h, *scratch):
            idx1_v, idx2_v, bufs, gsems, osems = scratch
            _sc_gather_body(c1, False, table_h, idx1_h, None, out1_h, None,
                            idx1_v, idx2_v, bufs, gsems, osems)

        out_type = jax.ShapeDtypeStruct((n1, _D), jnp.float32)

    k = pl.kernel(body, out_type=out_type, mesh=_MESH,
                  scratch_types=_scratch(c1))
    return k


_SC_KERNELS = tuple(
    _make_sc(c1, p == 0) for p, (_, c1) in enumerate(_PARTS))


def _tc_mlp_body(e3_ref, u_ref, w1t_ref, w1b_ref, w2_ref, w3t_ref,
                 b1_ref, b2_ref, b3_ref, out_ref):
    e3 = e3_ref[...]
    e2 = e3.reshape(_BB * _K, _D).astype(jnp.bfloat16)
    u = u_ref[...].astype(jnp.bfloat16)

    uw = jnp.dot(u, w1b_ref[...], preferred_element_type=jnp.float32)
    uw = uw + b1_ref[...]
    z1 = jnp.dot(e2, w1t_ref[...], preferred_element_type=jnp.float32)
    h1 = jnp.maximum(z1.reshape(_BB, _K, _D) + uw[:, None, :], 0.0)

    h2 = jnp.dot(h1.reshape(_BB * _K, _D).astype(jnp.bfloat16), w2_ref[...],
                 preferred_element_type=jnp.float32)
    h2 = jnp.maximum(h2 + b2_ref[...], 0.0)

    w3row = w3t_ref[...].reshape(1, 1, _D)
    t = jnp.sum(h2.reshape(_BB, _K, _D) * w3row, axis=2, keepdims=True)
    t = t + b3_ref[0, 0]

    m = jnp.max(t, axis=1, keepdims=True)
    p = jnp.exp(t - m)
    s = jnp.sum(p, axis=1, keepdims=True)
    att = p / s

    out_ref[...] = jnp.sum(e3 * att, axis=1)


def _tc_mlp(nodes_p, ublk, e3, u, w1t, w1b, w2, w3t, b1, b2, b3):
    return pl.pallas_call(
        _tc_mlp_body,
        grid=(nodes_p // _BB,),
        in_specs=[
            pl.BlockSpec((_BB, _K, _D), lambda i: (i, 0, 0)),
            pl.BlockSpec((_BB, _D), lambda i: (i + ublk, 0)),
            pl.BlockSpec((_D, _D), lambda i: (0, 0)),
            pl.BlockSpec((_D, _D), lambda i: (0, 0)),
            pl.BlockSpec((_D, _D), lambda i: (0, 0)),
            pl.BlockSpec((1, _D), lambda i: (0, 0)),
            pl.BlockSpec((1, _D), lambda i: (0, 0)),
            pl.BlockSpec((1, _D), lambda i: (0, 0)),
            pl.BlockSpec((1, 1), lambda i: (0, 0)),
        ],
        out_specs=pl.BlockSpec((_BB, _D), lambda i: (i, 0)),
        out_shape=jax.ShapeDtypeStruct((nodes_p, _D), jnp.float32),
    )(e3, u, w1t, w1b, w2, w3t, b1, b2, b3)


def kernel(nodes, to_neighs, u2e, W1, b1, W2, b2, W3, b3):
    nflat = to_neighs.reshape(-1)
    idx2 = jnp.zeros((_N2_PAD,), jnp.int32).at[:_B].set(
        nodes).reshape(_NW, _C2, _CH)

    idx1 = []
    row0 = 0
    for nodes_p, c1 in _PARTS:
        n1 = _NW * c1 * _CH
        nrows = nodes_p * _K
        idx1.append(jnp.zeros((n1,), jnp.int32).at[:nrows].set(
            nflat[row0: row0 + nrows]).reshape(_NW, c1, _CH))
        row0 += nrows

    e_parts = []
    for p, k in enumerate(_SC_KERNELS):
        if p == 0:
            ea, u_rows = k(u2e, idx1[0], idx2)
            e_parts.append(ea)
        else:
            e_parts.append(k(u2e, idx1[p]))

    bf = jnp.bfloat16
    args = (W1[:_D].astype(bf), W1[_D:].astype(bf), W2.astype(bf),
            W3.reshape(1, _D), b1.reshape(1, _D), b2.reshape(1, _D),
            b3.reshape(1, 1))
    outs = []
    ublk = 0
    for (nodes_p, c1), rows in zip(_PARTS, e_parts):
        e3 = rows.reshape(rows.shape[0] // _K, _K, _D)
        outs.append(_tc_mlp(nodes_p, ublk, e3, u_rows, *args))
        ublk += nodes_p // _BB
    return jnp.concatenate(outs, axis=0)
